# Initial kernel scaffold; baseline (speedup 1.0000x reference)
import jax
import jax.numpy as jnp
from jax.experimental import pallas as pl


def kernel(x, edge_index, batch, W1_0, b1_0, W2_0, b2_0, W1_1, b1_1, W2_1, b2_1, W1_2, b1_2, W2_2, b2_2, Wf, bf):
    raise NotImplementedError("write your pallas kernel here")


# speedup vs baseline: 1.0172x; 1.0172x over previous
import jax
import jax.numpy as jnp
from jax.experimental import pallas as pl

N = 100000
H = 128
G = 512
BN = 2000


def _gelu(v):
    return 0.5 * v * (1.0 + jax.lax.erf(v * 0.7071067811865476))


def _mlp_body(h_ref, agg_ref, w1_ref, b1_ref, w2_ref, b2_ref, o_ref):
    u = h_ref[...] + agg_ref[...]
    t = _gelu(jnp.dot(u, w1_ref[...], preferred_element_type=jnp.float32) + b1_ref[...])
    o_ref[...] = _gelu(jnp.dot(t, w2_ref[...], preferred_element_type=jnp.float32) + b2_ref[...])


def _mlp(h, agg, W1, b1, W2, b2):
    n, d = h.shape
    return pl.pallas_call(
        _mlp_body,
        grid=(n // BN,),
        in_specs=[
            pl.BlockSpec((BN, d), lambda i: (i, 0)),
            pl.BlockSpec((BN, d), lambda i: (i, 0)),
            pl.BlockSpec((d, H), lambda i: (0, 0)),
            pl.BlockSpec((1, H), lambda i: (0, 0)),
            pl.BlockSpec((H, H), lambda i: (0, 0)),
            pl.BlockSpec((1, H), lambda i: (0, 0)),
        ],
        out_specs=pl.BlockSpec((BN, H), lambda i: (i, 0)),
        out_shape=jax.ShapeDtypeStruct((n, H), jnp.float32),
    )(h, agg, W1, b1.reshape(1, H), W2, b2.reshape(1, H))


def _mlp0_body(u_ref, w1_ref, b1_ref, w2_ref, b2_ref, o_ref):
    t = _gelu(u_ref[...] * w1_ref[...] + b1_ref[...])
    o_ref[...] = _gelu(jnp.dot(t, w2_ref[...], preferred_element_type=jnp.float32) + b2_ref[...])


def _mlp0(u, W1, b1, W2, b2):
    n = u.shape[0]
    return pl.pallas_call(
        _mlp0_body,
        grid=(n // BN,),
        in_specs=[
            pl.BlockSpec((BN, 1), lambda i: (i, 0)),
            pl.BlockSpec((1, H), lambda i: (0, 0)),
            pl.BlockSpec((1, H), lambda i: (0, 0)),
            pl.BlockSpec((H, H), lambda i: (0, 0)),
            pl.BlockSpec((1, H), lambda i: (0, 0)),
        ],
        out_specs=pl.BlockSpec((BN, H), lambda i: (i, 0)),
        out_shape=jax.ShapeDtypeStruct((n, H), jnp.float32),
    )(u, W1, b1.reshape(1, H), W2, b2.reshape(1, H))


def _final_body(h_ref, batch_ref, wf_ref, bf_ref, o_ref):
    i = pl.program_id(0)

    @pl.when(i == 0)
    def _():
        o_ref[...] = jnp.zeros_like(o_ref)

    out = jnp.dot(h_ref[...], wf_ref[...], preferred_element_type=jnp.float32) + bf_ref[0, 0]
    gid = jax.lax.broadcasted_iota(jnp.int32, (1, G), 1)
    onehot = (batch_ref[...] == gid).astype(jnp.float32)
    o_ref[...] += jnp.sum(onehot * out, axis=0, keepdims=True)


def _final_pool(h, batch, Wf, bf):
    n = h.shape[0]
    pooled = pl.pallas_call(
        _final_body,
        grid=(n // BN,),
        in_specs=[
            pl.BlockSpec((BN, H), lambda i: (i, 0)),
            pl.BlockSpec((BN, 1), lambda i: (i, 0)),
            pl.BlockSpec((H, 1), lambda i: (0, 0)),
            pl.BlockSpec((1, 1), lambda i: (0, 0)),
        ],
        out_specs=pl.BlockSpec((1, G), lambda i: (0, 0)),
        out_shape=jax.ShapeDtypeStruct((1, G), jnp.float32),
    )(h, batch.reshape(n, 1), Wf, bf.reshape(1, 1))
    return pooled.reshape(G, 1)


def kernel(x, edge_index, batch,
           W1_0, b1_0, W2_0, b2_0,
           W1_1, b1_1, W2_1, b2_1,
           W1_2, b1_2, W2_2, b2_2,
           Wf, bf):
    src = edge_index[0]
    dst = edge_index[1]

    agg0 = jnp.zeros_like(x).at[dst].add(x[src])
    h = _mlp0(x + agg0, W1_0, b1_0, W2_0, b2_0)

    for (W1, b1, W2, b2) in ((W1_1, b1_1, W2_1, b2_1), (W1_2, b1_2, W2_2, b2_2)):
        agg = jnp.zeros_like(h).at[dst].add(h[src])
        h = _mlp(h, agg, W1, b1, W2, b2)

    return _final_pool(h, batch, Wf, bf)


# speedup vs baseline: 6.9474x; 6.8300x over previous
import functools

import jax
import jax.numpy as jnp
from jax import lax
from jax.experimental import pallas as pl
from jax.experimental.pallas import tpu as pltpu
from jax.experimental.pallas import tpu_sc as plsc

N = 100000
E = 1600000
H = 128
G = 512
BN = 2000

NP = 102400
GW = 16
NG = 8
FS = 128
EB = 2048

_VMESH = dict(core_axis_name="c", subcore_axis_name="s")
_SC_PARAMS = pltpu.CompilerParams(use_tc_tiling_on_sc=False)


ET0 = E // 32
NBLK0 = ET0 // EB
TAIL0 = ET0 - NBLK0 * EB
TAIL0_PAD = 896


def _l0_body(src_hbm, dst_hbm, x_hbm, out0_hbm, out1_hbm,
             sbuf, dbuf, dfire, vals, zbuf, agg_sp, gsem, ssem):
    c = lax.axis_index("c")
    s = lax.axis_index("s")

    def _z(r, carry):
        zbuf[pl.ds(r * 16, 16)] = jnp.zeros((16,), jnp.float32)
        return carry
    lax.fori_loop(0, 6400 // 16, _z, 0)
    pltpu.sync_copy(zbuf, agg_sp.at[pl.ds(s * 6400, 6400)])
    plsc.subcore_barrier()

    tile_base = c * (E // 2) + s * ET0

    def process(ebase, n_edges, nf):
        pltpu.sync_copy(src_hbm.at[pl.ds(ebase, n_edges)],
                        sbuf.at[pl.ds(0, n_edges)])
        pltpu.sync_copy(dst_hbm.at[pl.ds(ebase, n_edges)],
                        dbuf.at[pl.ds(0, n_edges)])
        if n_edges < nf * FS:
            for k in range(n_edges // 16, (nf * FS) // 16):
                sbuf[pl.ds(k * 16, 16)] = jnp.zeros((16,), jnp.int32)
                dbuf[pl.ds(k * 16, 16)] = jnp.full((16,), NP, jnp.int32)

        def cp(k, carry):
            dv = dbuf[pl.ds(k * 16, 16)]
            dfire[k // 8, pl.ds((k % 8) * 16, 16)] = dv
            return carry
        lax.fori_loop(0, (nf * FS) // 16, cp, 0)

        cps = [pltpu.async_copy(x_hbm.at[sbuf.at[pl.ds(f * FS, FS)]],
                                vals.at[pl.ds(f * FS, FS)], gsem)
               for f in range(nf)]
        for d in cps:
            d.wait()
        scs = [pltpu.async_copy(vals.at[pl.ds(f * FS, FS)],
                                agg_sp.at[dfire.at[f]], ssem, add=True)
               for f in range(nf)]
        for d in scs:
            d.wait()

    def blk(b, carry):
        process(tile_base + b * EB, EB, EB // FS)
        return carry
    lax.fori_loop(0, NBLK0, blk, 0)
    process(tile_base + NBLK0 * EB, TAIL0, TAIL0_PAD // FS)

    plsc.subcore_barrier()

    @pl.when(c == 0)
    def _():
        pltpu.sync_copy(agg_sp.at[pl.ds(s * 6400, 6400)],
                        out0_hbm.at[pl.ds(s * 6400, 6400)])

    @pl.when(c == 1)
    def _():
        pltpu.sync_copy(agg_sp.at[pl.ds(s * 6400, 6400)],
                        out1_hbm.at[pl.ds(s * 6400, 6400)])


def _l0_agg(xf, src, dst):
    kfn = pl.kernel(
        _l0_body,
        out_type=[jax.ShapeDtypeStruct((NP,), jnp.float32),
                  jax.ShapeDtypeStruct((NP,), jnp.float32)],
        mesh=plsc.VectorSubcoreMesh(**_VMESH),
        compiler_params=_SC_PARAMS,
        scratch_types=[
            pltpu.VMEM((EB,), jnp.int32),
            pltpu.VMEM((EB,), jnp.int32),
            pltpu.VMEM((16, FS), jnp.int32),
            pltpu.VMEM((EB,), jnp.float32),
            pltpu.VMEM((6400,), jnp.float32),
            pltpu.VMEM_SHARED((NP + FS,), jnp.float32),
            pltpu.SemaphoreType.DMA,
            pltpu.SemaphoreType.DMA,
        ],
    )
    return kfn(src, dst, xf)


TRB = 625


def _tr_body(h_hbm, ht_hbm, hin):
    c = lax.axis_index("c")
    s = lax.axis_index("s")
    w = s * 2 + c

    def blk(b, carry):
        r0 = w * (5 * TRB) + b * TRB
        pltpu.sync_copy(h_hbm.at[pl.ds(r0, TRB), :], hin)
        for g in range(NG):
            pltpu.sync_copy(hin.at[:, pl.ds(g * GW, GW)],
                            ht_hbm.at[pl.ds(g * NP + r0, TRB), :])
        return carry
    lax.fori_loop(0, 5, blk, 0)


def _transpose(h):
    kfn = pl.kernel(
        _tr_body,
        out_type=jax.ShapeDtypeStruct((NG * NP, GW), jnp.float32),
        mesh=plsc.VectorSubcoreMesh(**_VMESH),
        compiler_params=_SC_PARAMS,
        scratch_types=[
            pltpu.VMEM((TRB, H), jnp.float32),
        ],
    )
    return kfn(h)


EBA = 1024
ET = E // 16
NBLK = ET // EBA
TAIL = ET - NBLK * EBA
TAIL_PAD = 768


def _agg_body(src_hbm, dst_hbm, ht_hbm, agg_hbm,
              sbuf, dbuf, sidx, dfire, rows, zbuf, agg_sp, gsem, ssem):
    c = lax.axis_index("c")
    s = lax.axis_index("s")

    def _z(r, carry):
        zbuf[r, :] = jnp.zeros((GW,), jnp.float32)
        return carry
    lax.fori_loop(0, 400, _z, 0)

    tile_base = s * ET

    for gk in range(NG // 2):
        g = c * (NG // 2) + gk
        goff = g * NP

        for m in range(16):
            pltpu.sync_copy(zbuf, agg_sp.at[pl.ds(s * 6400 + m * 400, 400), :])
        plsc.subcore_barrier()

        def process(ebase, n_edges, nf):
            pltpu.sync_copy(src_hbm.at[pl.ds(ebase, n_edges)],
                            sbuf.at[pl.ds(0, n_edges)])
            pltpu.sync_copy(dst_hbm.at[pl.ds(ebase, n_edges)],
                            dbuf.at[pl.ds(0, n_edges)])
            if n_edges < nf * FS:
                for k in range(n_edges // 16, (nf * FS) // 16):
                    sbuf[pl.ds(k * 16, 16)] = jnp.zeros((16,), jnp.int32)
                    dbuf[pl.ds(k * 16, 16)] = jnp.full((16,), NP, jnp.int32)

            def cp(k, carry):
                sv = sbuf[pl.ds(k * 16, 16)] + goff
                sidx[pl.ds(k * 16, 16)] = sv
                dv = dbuf[pl.ds(k * 16, 16)]
                dfire[k // 8, pl.ds((k % 8) * 16, 16)] = dv
                return carry
            lax.fori_loop(0, (nf * FS) // 16, cp, 0)

            cps = [pltpu.async_copy(ht_hbm.at[sidx.at[pl.ds(f * FS, FS)]],
                                    rows.at[pl.ds(f * FS, FS), :], gsem)
                   for f in range(nf)]
            for d in cps:
                d.wait()
            scs = [pltpu.async_copy(rows.at[pl.ds(f * FS, FS), :],
                                    agg_sp.at[dfire.at[f]], ssem, add=True)
                   for f in range(nf)]
            for d in scs:
                d.wait()

        def blk(b, carry):
            process(tile_base + b * EBA, EBA, EBA // FS)
            return carry
        lax.fori_loop(0, NBLK, blk, 0)
        process(tile_base + NBLK * EBA, TAIL, TAIL_PAD // FS)

        plsc.subcore_barrier()
        pltpu.sync_copy(agg_sp.at[pl.ds(s * 6400, 6400), :],
                        agg_hbm.at[pl.ds(s * 6400, 6400), pl.ds(g * GW, GW)])
        plsc.subcore_barrier()


def _edge_agg(src, dst, ht):
    kfn = pl.kernel(
        _agg_body,
        out_type=jax.ShapeDtypeStruct((NP, H), jnp.float32),
        mesh=plsc.VectorSubcoreMesh(**_VMESH),
        compiler_params=_SC_PARAMS,
        scratch_types=[
            pltpu.VMEM((EBA,), jnp.int32),
            pltpu.VMEM((EBA,), jnp.int32),
            pltpu.VMEM((EBA,), jnp.int32),
            pltpu.VMEM((8, FS), jnp.int32),
            pltpu.VMEM((EBA, GW), jnp.float32),
            pltpu.VMEM((400, GW), jnp.float32),
            pltpu.VMEM_SHARED((NP + 8, GW), jnp.float32),
            pltpu.SemaphoreType.DMA,
            pltpu.SemaphoreType.DMA,
        ],
    )
    return kfn(src, dst, ht)


def _gelu(v):
    return 0.5 * v * (1.0 + jax.lax.erf(v * 0.7071067811865476))


def _mlp_body(h_ref, agg_ref, w1_ref, b1_ref, w2_ref, b2_ref, o_ref):
    u = h_ref[...] + agg_ref[...]
    t = _gelu(jnp.dot(u, w1_ref[...], preferred_element_type=jnp.float32) + b1_ref[...])
    o_ref[...] = _gelu(jnp.dot(t, w2_ref[...], preferred_element_type=jnp.float32) + b2_ref[...])


def _mlp(h, agg, W1, b1, W2, b2):
    n, d = h.shape
    return pl.pallas_call(
        _mlp_body,
        grid=(n // BN,),
        in_specs=[
            pl.BlockSpec((BN, d), lambda i: (i, 0)),
            pl.BlockSpec((BN, d), lambda i: (i, 0)),
            pl.BlockSpec((d, H), lambda i: (0, 0)),
            pl.BlockSpec((1, H), lambda i: (0, 0)),
            pl.BlockSpec((H, H), lambda i: (0, 0)),
            pl.BlockSpec((1, H), lambda i: (0, 0)),
        ],
        out_specs=pl.BlockSpec((BN, H), lambda i: (i, 0)),
        out_shape=jax.ShapeDtypeStruct((n, H), jnp.float32),
    )(h, agg, W1, b1.reshape(1, H), W2, b2.reshape(1, H))


def _mlp0_body(u_ref, w1_ref, b1_ref, w2_ref, b2_ref, o_ref):
    t = _gelu(u_ref[...] * w1_ref[...] + b1_ref[...])
    o_ref[...] = _gelu(jnp.dot(t, w2_ref[...], preferred_element_type=jnp.float32) + b2_ref[...])


def _mlp0(u, W1, b1, W2, b2):
    n = u.shape[0]
    return pl.pallas_call(
        _mlp0_body,
        grid=(n // BN,),
        in_specs=[
            pl.BlockSpec((BN, 1), lambda i: (i, 0)),
            pl.BlockSpec((1, H), lambda i: (0, 0)),
            pl.BlockSpec((1, H), lambda i: (0, 0)),
            pl.BlockSpec((H, H), lambda i: (0, 0)),
            pl.BlockSpec((1, H), lambda i: (0, 0)),
        ],
        out_specs=pl.BlockSpec((BN, H), lambda i: (i, 0)),
        out_shape=jax.ShapeDtypeStruct((n, H), jnp.float32),
    )(u, W1, b1.reshape(1, H), W2, b2.reshape(1, H))


def _final_body(h_ref, batch_ref, wf_ref, bf_ref, o_ref):
    i = pl.program_id(0)

    @pl.when(i == 0)
    def _():
        o_ref[...] = jnp.zeros_like(o_ref)

    out = jnp.dot(h_ref[...], wf_ref[...], preferred_element_type=jnp.float32) + bf_ref[0, 0]
    gid = jax.lax.broadcasted_iota(jnp.int32, (1, G), 1)
    onehot = (batch_ref[...] == gid).astype(jnp.float32)
    o_ref[...] += jnp.sum(onehot * out, axis=0, keepdims=True)


def _final_pool(h, batch, Wf, bf):
    n = h.shape[0]
    pooled = pl.pallas_call(
        _final_body,
        grid=(n // BN,),
        in_specs=[
            pl.BlockSpec((BN, H), lambda i: (i, 0)),
            pl.BlockSpec((BN, 1), lambda i: (i, 0)),
            pl.BlockSpec((H, 1), lambda i: (0, 0)),
            pl.BlockSpec((1, 1), lambda i: (0, 0)),
        ],
        out_specs=pl.BlockSpec((1, G), lambda i: (0, 0)),
        out_shape=jax.ShapeDtypeStruct((1, G), jnp.float32),
    )(h, batch.reshape(n, 1), Wf, bf.reshape(1, 1))
    return pooled.reshape(G, 1)


def kernel(x, edge_index, batch,
           W1_0, b1_0, W2_0, b2_0,
           W1_1, b1_1, W2_1, b2_1,
           W1_2, b1_2, W2_2, b2_2,
           Wf, bf):
    src = edge_index[0]
    dst = edge_index[1]
    xf = x.reshape(-1)

    p0, p1 = _l0_agg(xf, src, dst)
    u = x + (p0 + p1)[:N].reshape(N, 1)
    h = _mlp0(u, W1_0, b1_0, W2_0, b2_0)

    for (W1, b1, W2, b2) in ((W1_1, b1_1, W2_1, b2_1), (W1_2, b1_2, W2_2, b2_2)):
        ht = _transpose(h)
        agg = _edge_agg(src, dst, ht)
        h = _mlp(h, agg, W1, b1, W2, b2)

    return _final_pool(h, batch, Wf, bf)


# speedup vs baseline: 9.9257x; 1.4287x over previous
import functools

import jax
import jax.numpy as jnp
from jax import lax
from jax.experimental import pallas as pl
from jax.experimental.pallas import tpu as pltpu
from jax.experimental.pallas import tpu_sc as plsc

N = 100000
E = 1600000
H = 128
G = 512
BN = 2000

NP = 102400
GW = 16
NG = 8
FS = 128
EB = 2048

_VMESH = dict(core_axis_name="c", subcore_axis_name="s")
_SC_PARAMS = pltpu.CompilerParams(use_tc_tiling_on_sc=False)


ET0 = E // 32
NBLK0 = ET0 // EB
TAIL0 = ET0 - NBLK0 * EB
TAIL0_PAD = 896


def _l0_body(src_hbm, dst_hbm, x_hbm, out0_hbm, out1_hbm,
             sbuf, dbuf, dfire, vals, zbuf, agg_sp, gsem, ssem):
    c = lax.axis_index("c")
    s = lax.axis_index("s")

    def _z(r, carry):
        zbuf[pl.ds(r * 16, 16)] = jnp.zeros((16,), jnp.float32)
        return carry
    lax.fori_loop(0, 6400 // 16, _z, 0)
    pltpu.sync_copy(zbuf, agg_sp.at[pl.ds(s * 6400, 6400)])
    plsc.subcore_barrier()

    tile_base = c * (E // 2) + s * ET0

    def process(ebase, n_edges, nf):
        pltpu.sync_copy(src_hbm.at[pl.ds(ebase, n_edges)],
                        sbuf.at[pl.ds(0, n_edges)])
        pltpu.sync_copy(dst_hbm.at[pl.ds(ebase, n_edges)],
                        dbuf.at[pl.ds(0, n_edges)])
        if n_edges < nf * FS:
            for k in range(n_edges // 16, (nf * FS) // 16):
                sbuf[pl.ds(k * 16, 16)] = jnp.zeros((16,), jnp.int32)
                dbuf[pl.ds(k * 16, 16)] = jnp.full((16,), NP, jnp.int32)

        def cp(k, carry):
            dv = dbuf[pl.ds(k * 16, 16)]
            dfire[k // 8, pl.ds((k % 8) * 16, 16)] = dv
            return carry
        lax.fori_loop(0, (nf * FS) // 16, cp, 0)

        cps = [pltpu.async_copy(x_hbm.at[sbuf.at[pl.ds(f * FS, FS)]],
                                vals.at[pl.ds(f * FS, FS)], gsem)
               for f in range(nf)]
        for d in cps:
            d.wait()
        scs = [pltpu.async_copy(vals.at[pl.ds(f * FS, FS)],
                                agg_sp.at[dfire.at[f]], ssem, add=True)
               for f in range(nf)]
        for d in scs:
            d.wait()

    def blk(b, carry):
        process(tile_base + b * EB, EB, EB // FS)
        return carry
    lax.fori_loop(0, NBLK0, blk, 0)
    process(tile_base + NBLK0 * EB, TAIL0, TAIL0_PAD // FS)

    plsc.subcore_barrier()

    @pl.when(c == 0)
    def _():
        pltpu.sync_copy(agg_sp.at[pl.ds(s * 6400, 6400)],
                        out0_hbm.at[pl.ds(s * 6400, 6400)])

    @pl.when(c == 1)
    def _():
        pltpu.sync_copy(agg_sp.at[pl.ds(s * 6400, 6400)],
                        out1_hbm.at[pl.ds(s * 6400, 6400)])


def _l0_agg(xf, src, dst):
    kfn = pl.kernel(
        _l0_body,
        out_type=[jax.ShapeDtypeStruct((NP,), jnp.float32),
                  jax.ShapeDtypeStruct((NP,), jnp.float32)],
        mesh=plsc.VectorSubcoreMesh(**_VMESH),
        compiler_params=_SC_PARAMS,
        scratch_types=[
            pltpu.VMEM((EB,), jnp.int32),
            pltpu.VMEM((EB,), jnp.int32),
            pltpu.VMEM((16, FS), jnp.int32),
            pltpu.VMEM((EB,), jnp.float32),
            pltpu.VMEM((6400,), jnp.float32),
            pltpu.VMEM_SHARED((NP + FS,), jnp.float32),
            pltpu.SemaphoreType.DMA,
            pltpu.SemaphoreType.DMA,
        ],
    )
    return kfn(src, dst, xf)


TRB = 625


def _tr_body(h_hbm, ht_hbm, hin):
    c = lax.axis_index("c")
    s = lax.axis_index("s")
    w = s * 2 + c

    def blk(b, carry):
        r0 = w * (5 * TRB) + b * TRB
        pltpu.sync_copy(h_hbm.at[pl.ds(r0, TRB), :], hin)
        for g in range(NG):
            pltpu.sync_copy(hin.at[:, pl.ds(g * GW, GW)],
                            ht_hbm.at[pl.ds(g * NP + r0, TRB), :])
        return carry
    lax.fori_loop(0, 5, blk, 0)


def _transpose(h):
    kfn = pl.kernel(
        _tr_body,
        out_type=jax.ShapeDtypeStruct((NG * NP, GW), jnp.float32),
        mesh=plsc.VectorSubcoreMesh(**_VMESH),
        compiler_params=_SC_PARAMS,
        scratch_types=[
            pltpu.VMEM((TRB, H), jnp.float32),
        ],
    )
    return kfn(h)


EBA = 512
ET = E // 16
NBLK = ET // EBA
TAIL = ET - NBLK * EBA
TAIL_NF = 2


def _agg_body(src_hbm, dst_hbm, ht_hbm, agg_hbm,
              sbuf, dbuf, sidx, dfire, rows, zbuf, agg_sp, esem, gsem, ssem):
    c = lax.axis_index("c")
    s = lax.axis_index("s")

    def _z(r, carry):
        zbuf[r, :] = jnp.zeros((GW,), jnp.float32)
        return carry
    lax.fori_loop(0, 400, _z, 0)

    tile_base = s * ET
    NF = EBA // FS

    def issue_loads(b, q):
        ebase = tile_base + b * EBA
        pltpu.async_copy(src_hbm.at[pl.ds(ebase, EBA)], sbuf.at[q], esem)
        pltpu.async_copy(dst_hbm.at[pl.ds(ebase, EBA)], dbuf.at[q], esem)

    def wait_loads(q):
        for r in (sbuf, dbuf):
            pltpu.make_async_copy(src_hbm.at[pl.ds(tile_base, EBA)],
                                  r.at[q], esem).wait()

    def compute(q, goff, n16, pad16=None):
        if pad16 is not None:
            for k in range(pad16, n16):
                sbuf[q, pl.ds(k * 16, 16)] = jnp.zeros((16,), jnp.int32)
                dbuf[q, pl.ds(k * 16, 16)] = jnp.full((16,), NP, jnp.int32)

        def cp(k, carry):
            sidx[q, pl.ds(k * 16, 16)] = sbuf[q, pl.ds(k * 16, 16)] + goff
            dfire[q, k // 8, pl.ds((k % 8) * 16, 16)] = dbuf[q, pl.ds(k * 16, 16)]
            return carry
        lax.fori_loop(0, n16, cp, 0)

    def issue_gathers(q, nf):
        for f in range(nf):
            pltpu.async_copy(ht_hbm.at[sidx.at[q].at[pl.ds(f * FS, FS)]],
                             rows.at[q].at[pl.ds(f * FS, FS), :], gsem)

    def wait_gathers(q, nf):
        for f in range(nf):
            pltpu.make_async_copy(ht_hbm.at[sidx.at[q].at[pl.ds(f * FS, FS)]],
                                  rows.at[q].at[pl.ds(f * FS, FS), :],
                                  gsem).wait()

    def issue_scatters(q, nf):
        for f in range(nf):
            pltpu.async_copy(rows.at[q].at[pl.ds(f * FS, FS), :],
                             agg_sp.at[dfire.at[q].at[f]], ssem, add=True)

    def wait_scatters(q, nf):
        for f in range(nf):
            pltpu.make_async_copy(rows.at[q].at[pl.ds(f * FS, FS), :],
                                  agg_sp.at[dfire.at[q].at[f]], ssem).wait()

    for gk in range(NG // 2):
        g = c * (NG // 2) + gk
        goff = g * NP

        for m in range(16):
            pltpu.sync_copy(zbuf, agg_sp.at[pl.ds(s * 6400 + m * 400, 400), :])
        plsc.subcore_barrier()

        issue_loads(0, 0)
        wait_loads(0)
        compute(0, goff, EBA // 16)
        issue_gathers(0, NF)
        issue_loads(1, 1)

        wait_loads(1)
        compute(1, goff, EBA // 16)
        issue_loads(2, 0)
        wait_gathers(0, NF)
        issue_scatters(0, NF)
        issue_gathers(1, NF)

        def pair(i, carry):
            for (boff, q) in ((2, 0), (3, 1)):
                b = 2 * i + boff
                wait_loads(q)
                wait_scatters(q, NF)
                compute(q, goff, EBA // 16)
                issue_gathers(q, NF)
                issue_loads(b + 1, 1 - q)
                wait_gathers(1 - q, NF)
                issue_scatters(1 - q, NF)
            return carry
        lax.fori_loop(0, 96, pair, 0)

        wait_loads(0)
        wait_scatters(0, NF)
        compute(0, goff, EBA // 16)
        issue_gathers(0, NF)
        issue_loads(195, 1)
        wait_gathers(1, NF)
        issue_scatters(1, NF)

        wait_loads(1)
        wait_scatters(1, NF)
        compute(1, goff, TAIL_NF * FS // 16, pad16=TAIL // 16)
        issue_gathers(1, TAIL_NF)
        wait_gathers(0, NF)
        issue_scatters(0, NF)
        wait_gathers(1, TAIL_NF)
        issue_scatters(1, TAIL_NF)
        wait_scatters(0, NF)
        wait_scatters(1, TAIL_NF)

        plsc.subcore_barrier()
        pltpu.sync_copy(agg_sp.at[pl.ds(s * 6400, 6400), :],
                        agg_hbm.at[pl.ds(s * 6400, 6400), pl.ds(g * GW, GW)])
        plsc.subcore_barrier()


def _edge_agg(src, dst, ht):
    kfn = pl.kernel(
        _agg_body,
        out_type=jax.ShapeDtypeStruct((NP, H), jnp.float32),
        mesh=plsc.VectorSubcoreMesh(**_VMESH),
        compiler_params=_SC_PARAMS,
        scratch_types=[
            pltpu.VMEM((2, EBA), jnp.int32),
            pltpu.VMEM((2, EBA), jnp.int32),
            pltpu.VMEM((2, EBA), jnp.int32),
            pltpu.VMEM((2, EBA // FS, FS), jnp.int32),
            pltpu.VMEM((2, EBA, GW), jnp.float32),
            pltpu.VMEM((400, GW), jnp.float32),
            pltpu.VMEM_SHARED((NP + 8, GW), jnp.float32),
            pltpu.SemaphoreType.DMA,
            pltpu.SemaphoreType.DMA,
            pltpu.SemaphoreType.DMA,
        ],
    )
    return kfn(src, dst, ht)


def _gelu(v):
    return 0.5 * v * (1.0 + jax.lax.erf(v * 0.7071067811865476))


def _mlp_body(h_ref, agg_ref, w1_ref, b1_ref, w2_ref, b2_ref, o_ref):
    u = h_ref[...] + agg_ref[...]
    t = _gelu(jnp.dot(u, w1_ref[...], preferred_element_type=jnp.float32) + b1_ref[...])
    o_ref[...] = _gelu(jnp.dot(t, w2_ref[...], preferred_element_type=jnp.float32) + b2_ref[...])


def _mlp(h, agg, W1, b1, W2, b2):
    n, d = h.shape
    return pl.pallas_call(
        _mlp_body,
        grid=(n // BN,),
        in_specs=[
            pl.BlockSpec((BN, d), lambda i: (i, 0)),
            pl.BlockSpec((BN, d), lambda i: (i, 0)),
            pl.BlockSpec((d, H), lambda i: (0, 0)),
            pl.BlockSpec((1, H), lambda i: (0, 0)),
            pl.BlockSpec((H, H), lambda i: (0, 0)),
            pl.BlockSpec((1, H), lambda i: (0, 0)),
        ],
        out_specs=pl.BlockSpec((BN, H), lambda i: (i, 0)),
        out_shape=jax.ShapeDtypeStruct((n, H), jnp.float32),
    )(h, agg, W1, b1.reshape(1, H), W2, b2.reshape(1, H))


def _mlp0_body(u_ref, w1_ref, b1_ref, w2_ref, b2_ref, o_ref):
    t = _gelu(u_ref[...] * w1_ref[...] + b1_ref[...])
    o_ref[...] = _gelu(jnp.dot(t, w2_ref[...], preferred_element_type=jnp.float32) + b2_ref[...])


def _mlp0(u, W1, b1, W2, b2):
    n = u.shape[0]
    return pl.pallas_call(
        _mlp0_body,
        grid=(n // BN,),
        in_specs=[
            pl.BlockSpec((BN, 1), lambda i: (i, 0)),
            pl.BlockSpec((1, H), lambda i: (0, 0)),
            pl.BlockSpec((1, H), lambda i: (0, 0)),
            pl.BlockSpec((H, H), lambda i: (0, 0)),
            pl.BlockSpec((1, H), lambda i: (0, 0)),
        ],
        out_specs=pl.BlockSpec((BN, H), lambda i: (i, 0)),
        out_shape=jax.ShapeDtypeStruct((n, H), jnp.float32),
    )(u, W1, b1.reshape(1, H), W2, b2.reshape(1, H))


def _final_body(h_ref, batch_ref, wf_ref, bf_ref, o_ref):
    i = pl.program_id(0)

    @pl.when(i == 0)
    def _():
        o_ref[...] = jnp.zeros_like(o_ref)

    out = jnp.dot(h_ref[...], wf_ref[...], preferred_element_type=jnp.float32) + bf_ref[0, 0]
    gid = jax.lax.broadcasted_iota(jnp.int32, (1, G), 1)
    onehot = (batch_ref[...] == gid).astype(jnp.float32)
    o_ref[...] += jnp.sum(onehot * out, axis=0, keepdims=True)


def _final_pool(h, batch, Wf, bf):
    n = h.shape[0]
    pooled = pl.pallas_call(
        _final_body,
        grid=(n // BN,),
        in_specs=[
            pl.BlockSpec((BN, H), lambda i: (i, 0)),
            pl.BlockSpec((BN, 1), lambda i: (i, 0)),
            pl.BlockSpec((H, 1), lambda i: (0, 0)),
            pl.BlockSpec((1, 1), lambda i: (0, 0)),
        ],
        out_specs=pl.BlockSpec((1, G), lambda i: (0, 0)),
        out_shape=jax.ShapeDtypeStruct((1, G), jnp.float32),
    )(h, batch.reshape(n, 1), Wf, bf.reshape(1, 1))
    return pooled.reshape(G, 1)


def kernel(x, edge_index, batch,
           W1_0, b1_0, W2_0, b2_0,
           W1_1, b1_1, W2_1, b2_1,
           W1_2, b1_2, W2_2, b2_2,
           Wf, bf):
    src = edge_index[0]
    dst = edge_index[1]
    zpad = jnp.zeros((EBA,), jnp.int32)
    src_p = jnp.concatenate([src, zpad])
    dst_p = jnp.concatenate([dst, zpad])
    xf = x.reshape(-1)

    p0, p1 = _l0_agg(xf, src, dst)
    u = x + (p0 + p1)[:N].reshape(N, 1)
    h = _mlp0(u, W1_0, b1_0, W2_0, b2_0)

    for (W1, b1, W2, b2) in ((W1_1, b1_1, W2_1, b2_1), (W1_2, b1_2, W2_2, b2_2)):
        ht = _transpose(h)
        agg = _edge_agg(src_p, dst_p, ht)
        h = _mlp(h, agg, W1, b1, W2, b2)

    return _final_pool(h, batch, Wf, bf)


# speedup vs baseline: 9.9396x; 1.0014x over previous
import functools

import jax
import jax.numpy as jnp
from jax import lax
from jax.experimental import pallas as pl
from jax.experimental.pallas import tpu as pltpu
from jax.experimental.pallas import tpu_sc as plsc

N = 100000
E = 1600000
H = 128
G = 512
BN = 2000

NP = 102400
GW = 16
NG = 8
FS = 128
EB = 2048

_VMESH = dict(core_axis_name="c", subcore_axis_name="s")
_SC_PARAMS = pltpu.CompilerParams(use_tc_tiling_on_sc=False)


ET0 = E // 32
NBLK0 = ET0 // EB
TAIL0 = ET0 - NBLK0 * EB
TAIL0_PAD = 896


def _l0_body(src_hbm, dst_hbm, x_hbm, out0_hbm, out1_hbm,
             sbuf, dbuf, dfire, vals, zbuf, agg_sp, gsem, ssem):
    c = lax.axis_index("c")
    s = lax.axis_index("s")

    def _z(r, carry):
        zbuf[pl.ds(r * 16, 16)] = jnp.zeros((16,), jnp.float32)
        return carry
    lax.fori_loop(0, 6400 // 16, _z, 0)
    pltpu.sync_copy(zbuf, agg_sp.at[pl.ds(s * 6400, 6400)])
    plsc.subcore_barrier()

    tile_base = c * (E // 2) + s * ET0

    def process(ebase, n_edges, nf):
        pltpu.sync_copy(src_hbm.at[pl.ds(ebase, n_edges)],
                        sbuf.at[pl.ds(0, n_edges)])
        pltpu.sync_copy(dst_hbm.at[pl.ds(ebase, n_edges)],
                        dbuf.at[pl.ds(0, n_edges)])
        if n_edges < nf * FS:
            for k in range(n_edges // 16, (nf * FS) // 16):
                sbuf[pl.ds(k * 16, 16)] = jnp.zeros((16,), jnp.int32)
                dbuf[pl.ds(k * 16, 16)] = jnp.full((16,), NP, jnp.int32)

        def cp(k, carry):
            dv = dbuf[pl.ds(k * 16, 16)]
            dfire[k // 8, pl.ds((k % 8) * 16, 16)] = dv
            return carry
        lax.fori_loop(0, (nf * FS) // 16, cp, 0)

        cps = [pltpu.async_copy(x_hbm.at[sbuf.at[pl.ds(f * FS, FS)]],
                                vals.at[pl.ds(f * FS, FS)], gsem)
               for f in range(nf)]
        for d in cps:
            d.wait()
        scs = [pltpu.async_copy(vals.at[pl.ds(f * FS, FS)],
                                agg_sp.at[dfire.at[f]], ssem, add=True)
               for f in range(nf)]
        for d in scs:
            d.wait()

    def blk(b, carry):
        process(tile_base + b * EB, EB, EB // FS)
        return carry
    lax.fori_loop(0, NBLK0, blk, 0)
    process(tile_base + NBLK0 * EB, TAIL0, TAIL0_PAD // FS)

    plsc.subcore_barrier()

    @pl.when(c == 0)
    def _():
        pltpu.sync_copy(agg_sp.at[pl.ds(s * 6400, 6400)],
                        out0_hbm.at[pl.ds(s * 6400, 6400)])

    @pl.when(c == 1)
    def _():
        pltpu.sync_copy(agg_sp.at[pl.ds(s * 6400, 6400)],
                        out1_hbm.at[pl.ds(s * 6400, 6400)])


def _l0_agg(xf, src, dst):
    kfn = pl.kernel(
        _l0_body,
        out_type=[jax.ShapeDtypeStruct((NP,), jnp.float32),
                  jax.ShapeDtypeStruct((NP,), jnp.float32)],
        mesh=plsc.VectorSubcoreMesh(**_VMESH),
        compiler_params=_SC_PARAMS,
        scratch_types=[
            pltpu.VMEM((EB,), jnp.int32),
            pltpu.VMEM((EB,), jnp.int32),
            pltpu.VMEM((16, FS), jnp.int32),
            pltpu.VMEM((EB,), jnp.float32),
            pltpu.VMEM((6400,), jnp.float32),
            pltpu.VMEM_SHARED((NP + FS,), jnp.float32),
            pltpu.SemaphoreType.DMA,
            pltpu.SemaphoreType.DMA,
        ],
    )
    return kfn(src, dst, xf)


TRB = 625


def _tr_body(h_hbm, ht_hbm, hin):
    c = lax.axis_index("c")
    s = lax.axis_index("s")
    w = s * 2 + c

    def blk(b, carry):
        r0 = w * (5 * TRB) + b * TRB
        pltpu.sync_copy(h_hbm.at[pl.ds(r0, TRB), :], hin)
        for g in range(NG):
            pltpu.sync_copy(hin.at[:, pl.ds(g * GW, GW)],
                            ht_hbm.at[pl.ds(g * NP + r0, TRB), :])
        return carry
    lax.fori_loop(0, 5, blk, 0)


def _transpose(h):
    kfn = pl.kernel(
        _tr_body,
        out_type=jax.ShapeDtypeStruct((NG * NP, GW), jnp.float32),
        mesh=plsc.VectorSubcoreMesh(**_VMESH),
        compiler_params=_SC_PARAMS,
        scratch_types=[
            pltpu.VMEM((TRB, H), jnp.float32),
        ],
    )
    return kfn(h)


EBA = 512
NF = EBA // FS
NBLK = 196
ETP = NBLK * EBA
EP = 16 * ETP


def _agg_body(srcg_hbm, dst2_hbm, ht_hbm, agg_hbm,
              sbuf, dbuf, rows, zbuf, agg_sp, esem, gsem, ssem):
    c = lax.axis_index("c")
    s = lax.axis_index("s")

    def _z(r, carry):
        zbuf[r, :] = jnp.zeros((GW,), jnp.float32)
        return carry
    lax.fori_loop(0, 400, _z, 0)

    tile_base = s * ETP
    rowbase = s * (ETP // FS)

    def issue_loads(b, goff):
        q = b % 4
        pltpu.async_copy(srcg_hbm.at[pl.ds(goff + tile_base + b * EBA, EBA)],
                         sbuf.at[q], esem)
        pltpu.async_copy(dst2_hbm.at[pl.ds(rowbase + b * NF, NF), :],
                         dbuf.at[q], esem)

    def wait_loads(b):
        q = b % 4
        pltpu.make_async_copy(srcg_hbm.at[pl.ds(tile_base, EBA)],
                              sbuf.at[q], esem).wait()
        pltpu.make_async_copy(dst2_hbm.at[pl.ds(rowbase, NF), :],
                              dbuf.at[q], esem).wait()

    def issue_gathers(b):
        ql, qr = b % 4, b % 2
        for f in range(NF):
            pltpu.async_copy(ht_hbm.at[sbuf.at[ql].at[pl.ds(f * FS, FS)]],
                             rows.at[qr].at[pl.ds(f * FS, FS), :], gsem)

    def wait_gathers(b):
        ql, qr = b % 4, b % 2
        for f in range(NF):
            pltpu.make_async_copy(ht_hbm.at[sbuf.at[ql].at[pl.ds(f * FS, FS)]],
                                  rows.at[qr].at[pl.ds(f * FS, FS), :],
                                  gsem).wait()

    def issue_scatters(b):
        ql, qr = b % 4, b % 2
        for f in range(NF):
            pltpu.async_copy(rows.at[qr].at[pl.ds(f * FS, FS), :],
                             agg_sp.at[dbuf.at[ql].at[f]], ssem, add=True)

    def wait_scatters(b):
        ql, qr = b % 4, b % 2
        for f in range(NF):
            pltpu.make_async_copy(rows.at[qr].at[pl.ds(f * FS, FS), :],
                                  agg_sp.at[dbuf.at[ql].at[f]], ssem).wait()

    for gk in range(NG // 2):
        g = c * (NG // 2) + gk
        goff = g * EP

        for m in range(16):
            pltpu.sync_copy(zbuf, agg_sp.at[pl.ds(s * 6400 + m * 400, 400), :])
        plsc.subcore_barrier()

        issue_loads(0, goff)
        issue_loads(1, goff)
        wait_loads(0)
        issue_gathers(0)
        issue_loads(2, goff)
        wait_loads(1)
        issue_gathers(1)
        issue_loads(3, goff)
        wait_gathers(0)
        issue_scatters(0)

        def quad(i, carry):
            for j in range(4):
                b = 4 * i + 2 + j
                wait_loads(b)
                wait_scatters(b - 2)
                issue_gathers(b)
                issue_loads(b + 2, goff)
                wait_gathers(b - 1)
                issue_scatters(b - 1)
            return carry
        lax.fori_loop(0, 48, quad, 0)

        for b in (194, 195):
            wait_loads(b)
            wait_scatters(b - 2)
            issue_gathers(b)
            wait_gathers(b - 1)
            issue_scatters(b - 1)
        wait_gathers(195)
        issue_scatters(195)
        wait_scatters(194)
        wait_scatters(195)

        plsc.subcore_barrier()
        pltpu.sync_copy(agg_sp.at[pl.ds(s * 6400, 6400), :],
                        agg_hbm.at[pl.ds(s * 6400, 6400), pl.ds(g * GW, GW)])
        plsc.subcore_barrier()


def _edge_agg(srcg, dst2, ht):
    kfn = pl.kernel(
        _agg_body,
        out_type=jax.ShapeDtypeStruct((NP, H), jnp.float32),
        mesh=plsc.VectorSubcoreMesh(**_VMESH),
        compiler_params=_SC_PARAMS,
        scratch_types=[
            pltpu.VMEM((4, EBA), jnp.int32),
            pltpu.VMEM((4, NF, FS), jnp.int32),
            pltpu.VMEM((2, EBA, GW), jnp.float32),
            pltpu.VMEM((400, GW), jnp.float32),
            pltpu.VMEM_SHARED((NP + 8, GW), jnp.float32),
            pltpu.SemaphoreType.DMA,
            pltpu.SemaphoreType.DMA,
            pltpu.SemaphoreType.DMA,
        ],
    )
    return kfn(srcg, dst2, ht)


def _gelu(v):
    return 0.5 * v * (1.0 + jax.lax.erf(v * 0.7071067811865476))


def _mlp_body(h_ref, agg_ref, w1_ref, b1_ref, w2_ref, b2_ref, o_ref):
    u = h_ref[...] + agg_ref[...]
    t = _gelu(jnp.dot(u, w1_ref[...], preferred_element_type=jnp.float32) + b1_ref[...])
    o_ref[...] = _gelu(jnp.dot(t, w2_ref[...], preferred_element_type=jnp.float32) + b2_ref[...])


def _mlp(h, agg, W1, b1, W2, b2):
    n, d = h.shape
    return pl.pallas_call(
        _mlp_body,
        grid=(n // BN,),
        in_specs=[
            pl.BlockSpec((BN, d), lambda i: (i, 0)),
            pl.BlockSpec((BN, d), lambda i: (i, 0)),
            pl.BlockSpec((d, H), lambda i: (0, 0)),
            pl.BlockSpec((1, H), lambda i: (0, 0)),
            pl.BlockSpec((H, H), lambda i: (0, 0)),
            pl.BlockSpec((1, H), lambda i: (0, 0)),
        ],
        out_specs=pl.BlockSpec((BN, H), lambda i: (i, 0)),
        out_shape=jax.ShapeDtypeStruct((n, H), jnp.float32),
    )(h, agg, W1, b1.reshape(1, H), W2, b2.reshape(1, H))


def _mlp0_body(u_ref, w1_ref, b1_ref, w2_ref, b2_ref, o_ref):
    t = _gelu(u_ref[...] * w1_ref[...] + b1_ref[...])
    o_ref[...] = _gelu(jnp.dot(t, w2_ref[...], preferred_element_type=jnp.float32) + b2_ref[...])


def _mlp0(u, W1, b1, W2, b2):
    n = u.shape[0]
    return pl.pallas_call(
        _mlp0_body,
        grid=(n // BN,),
        in_specs=[
            pl.BlockSpec((BN, 1), lambda i: (i, 0)),
            pl.BlockSpec((1, H), lambda i: (0, 0)),
            pl.BlockSpec((1, H), lambda i: (0, 0)),
            pl.BlockSpec((H, H), lambda i: (0, 0)),
            pl.BlockSpec((1, H), lambda i: (0, 0)),
        ],
        out_specs=pl.BlockSpec((BN, H), lambda i: (i, 0)),
        out_shape=jax.ShapeDtypeStruct((n, H), jnp.float32),
    )(u, W1, b1.reshape(1, H), W2, b2.reshape(1, H))


def _final_body(h_ref, batch_ref, wf_ref, bf_ref, o_ref):
    i = pl.program_id(0)

    @pl.when(i == 0)
    def _():
        o_ref[...] = jnp.zeros_like(o_ref)

    out = jnp.dot(h_ref[...], wf_ref[...], preferred_element_type=jnp.float32) + bf_ref[0, 0]
    gid = jax.lax.broadcasted_iota(jnp.int32, (1, G), 1)
    onehot = (batch_ref[...] == gid).astype(jnp.float32)
    o_ref[...] += jnp.sum(onehot * out, axis=0, keepdims=True)


def _final_pool(h, batch, Wf, bf):
    n = h.shape[0]
    pooled = pl.pallas_call(
        _final_body,
        grid=(n // BN,),
        in_specs=[
            pl.BlockSpec((BN, H), lambda i: (i, 0)),
            pl.BlockSpec((BN, 1), lambda i: (i, 0)),
            pl.BlockSpec((H, 1), lambda i: (0, 0)),
            pl.BlockSpec((1, 1), lambda i: (0, 0)),
        ],
        out_specs=pl.BlockSpec((1, G), lambda i: (0, 0)),
        out_shape=jax.ShapeDtypeStruct((1, G), jnp.float32),
    )(h, batch.reshape(n, 1), Wf, bf.reshape(1, 1))
    return pooled.reshape(G, 1)


def kernel(x, edge_index, batch,
           W1_0, b1_0, W2_0, b2_0,
           W1_1, b1_1, W2_1, b2_1,
           W1_2, b1_2, W2_2, b2_2,
           Wf, bf):
    src = edge_index[0]
    dst = edge_index[1]
    pad = EP - E
    src_pp = jnp.concatenate([src, jnp.zeros((pad,), jnp.int32)])
    dst_pp = jnp.concatenate([dst, jnp.full((pad,), NP, jnp.int32)])
    srcg = (src_pp[None, :]
            + (jnp.arange(NG, dtype=jnp.int32) * NP)[:, None]).reshape(-1)
    dst2 = dst_pp.reshape(EP // FS, FS)
    xf = x.reshape(-1)

    p0, p1 = _l0_agg(xf, src, dst)
    u = x + (p0 + p1)[:N].reshape(N, 1)
    h = _mlp0(u, W1_0, b1_0, W2_0, b2_0)

    for (W1, b1, W2, b2) in ((W1_1, b1_1, W2_1, b2_1), (W1_2, b1_2, W2_2, b2_2)):
        ht = _transpose(h)
        agg = _edge_agg(srcg, dst2, ht)
        h = _mlp(h, agg, W1, b1, W2, b2)

    return _final_pool(h, batch, Wf, bf)


# speedup vs baseline: 9.9567x; 1.0017x over previous
import functools

import jax
import jax.numpy as jnp
from jax import lax
from jax.experimental import pallas as pl
from jax.experimental.pallas import tpu as pltpu
from jax.experimental.pallas import tpu_sc as plsc

N = 100000
E = 1600000
H = 128
G = 512
BN = 2000

NP = 102400
GW = 16
NG = 8
FS = 128
EB = 2048

_VMESH = dict(core_axis_name="c", subcore_axis_name="s")
_SC_PARAMS = pltpu.CompilerParams(use_tc_tiling_on_sc=False)


ET0 = E // 32
NBLK0 = ET0 // EB
TAIL0 = ET0 - NBLK0 * EB
TAIL0_PAD = 896


def _l0_body(src_hbm, dst_hbm, x_hbm, out0_hbm, out1_hbm,
             sbuf, dbuf, dfire, vals, zbuf, agg_sp, gsem, ssem):
    c = lax.axis_index("c")
    s = lax.axis_index("s")

    def _z(r, carry):
        zbuf[pl.ds(r * 16, 16)] = jnp.zeros((16,), jnp.float32)
        return carry
    lax.fori_loop(0, 6400 // 16, _z, 0)
    pltpu.sync_copy(zbuf, agg_sp.at[pl.ds(s * 6400, 6400)])
    plsc.subcore_barrier()

    tile_base = c * (E // 2) + s * ET0

    def process(ebase, n_edges, nf):
        pltpu.sync_copy(src_hbm.at[pl.ds(ebase, n_edges)],
                        sbuf.at[pl.ds(0, n_edges)])
        pltpu.sync_copy(dst_hbm.at[pl.ds(ebase, n_edges)],
                        dbuf.at[pl.ds(0, n_edges)])
        if n_edges < nf * FS:
            for k in range(n_edges // 16, (nf * FS) // 16):
                sbuf[pl.ds(k * 16, 16)] = jnp.zeros((16,), jnp.int32)
                dbuf[pl.ds(k * 16, 16)] = jnp.full((16,), NP, jnp.int32)

        def cp(k, carry):
            dv = dbuf[pl.ds(k * 16, 16)]
            dfire[k // 8, pl.ds((k % 8) * 16, 16)] = dv
            return carry
        lax.fori_loop(0, (nf * FS) // 16, cp, 0)

        cps = [pltpu.async_copy(x_hbm.at[sbuf.at[pl.ds(f * FS, FS)]],
                                vals.at[pl.ds(f * FS, FS)], gsem)
               for f in range(nf)]
        for d in cps:
            d.wait()
        scs = [pltpu.async_copy(vals.at[pl.ds(f * FS, FS)],
                                agg_sp.at[dfire.at[f]], ssem, add=True)
               for f in range(nf)]
        for d in scs:
            d.wait()

    def blk(b, carry):
        process(tile_base + b * EB, EB, EB // FS)
        return carry
    lax.fori_loop(0, NBLK0, blk, 0)
    process(tile_base + NBLK0 * EB, TAIL0, TAIL0_PAD // FS)

    plsc.subcore_barrier()

    @pl.when(c == 0)
    def _():
        pltpu.sync_copy(agg_sp.at[pl.ds(s * 6400, 6400)],
                        out0_hbm.at[pl.ds(s * 6400, 6400)])

    @pl.when(c == 1)
    def _():
        pltpu.sync_copy(agg_sp.at[pl.ds(s * 6400, 6400)],
                        out1_hbm.at[pl.ds(s * 6400, 6400)])


def _l0_agg(xf, src, dst):
    kfn = pl.kernel(
        _l0_body,
        out_type=[jax.ShapeDtypeStruct((NP,), jnp.float32),
                  jax.ShapeDtypeStruct((NP,), jnp.float32)],
        mesh=plsc.VectorSubcoreMesh(**_VMESH),
        compiler_params=_SC_PARAMS,
        scratch_types=[
            pltpu.VMEM((EB,), jnp.int32),
            pltpu.VMEM((EB,), jnp.int32),
            pltpu.VMEM((16, FS), jnp.int32),
            pltpu.VMEM((EB,), jnp.float32),
            pltpu.VMEM((6400,), jnp.float32),
            pltpu.VMEM_SHARED((NP + FS,), jnp.float32),
            pltpu.SemaphoreType.DMA,
            pltpu.SemaphoreType.DMA,
        ],
    )
    return kfn(src, dst, xf)


TRB = 625


def _tr_body(h_hbm, ht_hbm, hin):
    c = lax.axis_index("c")
    s = lax.axis_index("s")
    w = s * 2 + c

    def blk(b, carry):
        r0 = w * (5 * TRB) + b * TRB
        pltpu.sync_copy(h_hbm.at[pl.ds(r0, TRB), :], hin)
        for g in range(NG):
            pltpu.sync_copy(hin.at[:, pl.ds(g * GW, GW)],
                            ht_hbm.at[pl.ds(g * NP + r0, TRB), :])
        return carry
    lax.fori_loop(0, 5, blk, 0)


def _transpose(h):
    kfn = pl.kernel(
        _tr_body,
        out_type=jax.ShapeDtypeStruct((NG * NP, GW), jnp.float32),
        mesh=plsc.VectorSubcoreMesh(**_VMESH),
        compiler_params=_SC_PARAMS,
        scratch_types=[
            pltpu.VMEM((TRB, H), jnp.float32),
        ],
    )
    return kfn(h)


EBA = 512
NF = EBA // FS
NBLK = 196
ETP = NBLK * EBA
EP = 16 * ETP


def _agg_body(srcg_hbm, dst2_hbm, ht_hbm, agg_hbm,
              sbuf, dbuf, rows, zbuf, agg_sp, esem, gsem, ssem):
    c = lax.axis_index("c")
    s = lax.axis_index("s")

    def _z(r, carry):
        zbuf[r, :] = jnp.zeros((GW,), jnp.float32)
        return carry
    lax.fori_loop(0, 400, _z, 0)

    tile_base = s * ETP

    def issue_loads(b, goff):
        q = b % 4
        pltpu.async_copy(srcg_hbm.at[pl.ds(goff + tile_base + b * EBA, EBA)],
                         sbuf.at[q], esem)
        pltpu.async_copy(dst2_hbm.at[pl.ds(tile_base + b * EBA, EBA)],
                         dbuf.at[q], esem)

    def wait_loads(b):
        q = b % 4
        pltpu.make_async_copy(srcg_hbm.at[pl.ds(tile_base, EBA)],
                              sbuf.at[q], esem).wait()
        pltpu.make_async_copy(dst2_hbm.at[pl.ds(tile_base, EBA)],
                              dbuf.at[q], esem).wait()

    def issue_gathers(b):
        ql, qr = b % 4, b % 2
        pltpu.async_copy(ht_hbm.at[sbuf.at[ql]], rows.at[qr], gsem)

    def wait_gathers(b):
        ql, qr = b % 4, b % 2
        pltpu.make_async_copy(ht_hbm.at[sbuf.at[ql]], rows.at[qr], gsem).wait()

    def issue_scatters(b):
        ql, qr = b % 4, b % 2
        pltpu.async_copy(rows.at[qr], agg_sp.at[dbuf.at[ql]], ssem, add=True)

    def wait_scatters(b):
        ql, qr = b % 4, b % 2
        pltpu.make_async_copy(rows.at[qr], agg_sp.at[dbuf.at[ql]], ssem).wait()

    for gk in range(NG // 2):
        g = c * (NG // 2) + gk
        goff = g * EP

        for m in range(16):
            pltpu.sync_copy(zbuf, agg_sp.at[pl.ds(s * 6400 + m * 400, 400), :])
        plsc.subcore_barrier()

        issue_loads(0, goff)
        issue_loads(1, goff)
        wait_loads(0)
        issue_gathers(0)
        issue_loads(2, goff)
        wait_loads(1)
        issue_gathers(1)
        issue_loads(3, goff)
        wait_gathers(0)
        issue_scatters(0)

        def quad(i, carry):
            for j in range(4):
                b = 4 * i + 2 + j
                wait_loads(b)
                wait_scatters(b - 2)
                issue_gathers(b)
                issue_loads(b + 2, goff)
                wait_gathers(b - 1)
                issue_scatters(b - 1)
            return carry
        lax.fori_loop(0, 48, quad, 0)

        for b in (194, 195):
            wait_loads(b)
            wait_scatters(b - 2)
            issue_gathers(b)
            wait_gathers(b - 1)
            issue_scatters(b - 1)
        wait_gathers(195)
        issue_scatters(195)
        wait_scatters(194)
        wait_scatters(195)

        plsc.subcore_barrier()
        pltpu.sync_copy(agg_sp.at[pl.ds(s * 6400, 6400), :],
                        agg_hbm.at[pl.ds(s * 6400, 6400), pl.ds(g * GW, GW)])
        plsc.subcore_barrier()


def _edge_agg(srcg, dst2, ht):
    kfn = pl.kernel(
        _agg_body,
        out_type=jax.ShapeDtypeStruct((NP, H), jnp.float32),
        mesh=plsc.VectorSubcoreMesh(**_VMESH),
        compiler_params=_SC_PARAMS,
        scratch_types=[
            pltpu.VMEM((4, EBA), jnp.int32),
            pltpu.VMEM((4, EBA), jnp.int32),
            pltpu.VMEM((2, EBA, GW), jnp.float32),
            pltpu.VMEM((400, GW), jnp.float32),
            pltpu.VMEM_SHARED((NP + 8, GW), jnp.float32),
            pltpu.SemaphoreType.DMA,
            pltpu.SemaphoreType.DMA,
            pltpu.SemaphoreType.DMA,
        ],
    )
    return kfn(srcg, dst2, ht)


def _gelu(v):
    return 0.5 * v * (1.0 + jax.lax.erf(v * 0.7071067811865476))


def _mlp_body(h_ref, agg_ref, w1_ref, b1_ref, w2_ref, b2_ref, o_ref):
    u = h_ref[...] + agg_ref[...]
    t = _gelu(jnp.dot(u, w1_ref[...], preferred_element_type=jnp.float32) + b1_ref[...])
    o_ref[...] = _gelu(jnp.dot(t, w2_ref[...], preferred_element_type=jnp.float32) + b2_ref[...])


def _mlp(h, agg, W1, b1, W2, b2):
    n, d = h.shape
    return pl.pallas_call(
        _mlp_body,
        grid=(n // BN,),
        in_specs=[
            pl.BlockSpec((BN, d), lambda i: (i, 0)),
            pl.BlockSpec((BN, d), lambda i: (i, 0)),
            pl.BlockSpec((d, H), lambda i: (0, 0)),
            pl.BlockSpec((1, H), lambda i: (0, 0)),
            pl.BlockSpec((H, H), lambda i: (0, 0)),
            pl.BlockSpec((1, H), lambda i: (0, 0)),
        ],
        out_specs=pl.BlockSpec((BN, H), lambda i: (i, 0)),
        out_shape=jax.ShapeDtypeStruct((n, H), jnp.float32),
    )(h, agg, W1, b1.reshape(1, H), W2, b2.reshape(1, H))


def _mlp0_body(u_ref, w1_ref, b1_ref, w2_ref, b2_ref, o_ref):
    t = _gelu(u_ref[...] * w1_ref[...] + b1_ref[...])
    o_ref[...] = _gelu(jnp.dot(t, w2_ref[...], preferred_element_type=jnp.float32) + b2_ref[...])


def _mlp0(u, W1, b1, W2, b2):
    n = u.shape[0]
    return pl.pallas_call(
        _mlp0_body,
        grid=(n // BN,),
        in_specs=[
            pl.BlockSpec((BN, 1), lambda i: (i, 0)),
            pl.BlockSpec((1, H), lambda i: (0, 0)),
            pl.BlockSpec((1, H), lambda i: (0, 0)),
            pl.BlockSpec((H, H), lambda i: (0, 0)),
            pl.BlockSpec((1, H), lambda i: (0, 0)),
        ],
        out_specs=pl.BlockSpec((BN, H), lambda i: (i, 0)),
        out_shape=jax.ShapeDtypeStruct((n, H), jnp.float32),
    )(u, W1, b1.reshape(1, H), W2, b2.reshape(1, H))


def _final_body(h_ref, batch_ref, wf_ref, bf_ref, o_ref):
    i = pl.program_id(0)

    @pl.when(i == 0)
    def _():
        o_ref[...] = jnp.zeros_like(o_ref)

    out = jnp.dot(h_ref[...], wf_ref[...], preferred_element_type=jnp.float32) + bf_ref[0, 0]
    gid = jax.lax.broadcasted_iota(jnp.int32, (1, G), 1)
    onehot = (batch_ref[...] == gid).astype(jnp.float32)
    o_ref[...] += jnp.sum(onehot * out, axis=0, keepdims=True)


def _final_pool(h, batch, Wf, bf):
    n = h.shape[0]
    pooled = pl.pallas_call(
        _final_body,
        grid=(n // BN,),
        in_specs=[
            pl.BlockSpec((BN, H), lambda i: (i, 0)),
            pl.BlockSpec((BN, 1), lambda i: (i, 0)),
            pl.BlockSpec((H, 1), lambda i: (0, 0)),
            pl.BlockSpec((1, 1), lambda i: (0, 0)),
        ],
        out_specs=pl.BlockSpec((1, G), lambda i: (0, 0)),
        out_shape=jax.ShapeDtypeStruct((1, G), jnp.float32),
    )(h, batch.reshape(n, 1), Wf, bf.reshape(1, 1))
    return pooled.reshape(G, 1)


def kernel(x, edge_index, batch,
           W1_0, b1_0, W2_0, b2_0,
           W1_1, b1_1, W2_1, b2_1,
           W1_2, b1_2, W2_2, b2_2,
           Wf, bf):
    src = edge_index[0]
    dst = edge_index[1]
    pad = EP - E
    src_pp = jnp.concatenate([src, jnp.zeros((pad,), jnp.int32)])
    dst_pp = jnp.concatenate([dst, jnp.full((pad,), NP, jnp.int32)])
    srcg = (src_pp[None, :]
            + (jnp.arange(NG, dtype=jnp.int32) * NP)[:, None]).reshape(-1)
    dst2 = dst_pp
    xf = x.reshape(-1)

    p0, p1 = _l0_agg(xf, src, dst)
    u = x + (p0 + p1)[:N].reshape(N, 1)
    h = _mlp0(u, W1_0, b1_0, W2_0, b2_0)

    for (W1, b1, W2, b2) in ((W1_1, b1_1, W2_1, b2_1), (W1_2, b1_2, W2_2, b2_2)):
        ht = _transpose(h)
        agg = _edge_agg(srcg, dst2, ht)
        h = _mlp(h, agg, W1, b1, W2, b2)

    return _final_pool(h, batch, Wf, bf)


# speedup vs baseline: 10.0735x; 1.0117x over previous
import functools

import jax
import jax.numpy as jnp
from jax import lax
from jax.experimental import pallas as pl
from jax.experimental.pallas import tpu as pltpu
from jax.experimental.pallas import tpu_sc as plsc

N = 100000
E = 1600000
H = 128
G = 512
BN = 2000

NP = 102400
GW = 16
NG = 8
FS = 128
EB = 2048

_VMESH = dict(core_axis_name="c", subcore_axis_name="s")
_SC_PARAMS = pltpu.CompilerParams(use_tc_tiling_on_sc=False)


ET0 = E // 32
NBLK0 = ET0 // EB
TAIL0 = ET0 - NBLK0 * EB
TAIL0_PAD = 896


def _l0_body(src_hbm, dst_hbm, x_hbm, out0_hbm, out1_hbm,
             sbuf, dbuf, dfire, vals, zbuf, agg_sp, gsem, ssem):
    c = lax.axis_index("c")
    s = lax.axis_index("s")

    def _z(r, carry):
        zbuf[pl.ds(r * 16, 16)] = jnp.zeros((16,), jnp.float32)
        return carry
    lax.fori_loop(0, 6400 // 16, _z, 0)
    pltpu.sync_copy(zbuf, agg_sp.at[pl.ds(s * 6400, 6400)])
    plsc.subcore_barrier()

    tile_base = c * (E // 2) + s * ET0

    def process(ebase, n_edges, nf):
        pltpu.sync_copy(src_hbm.at[pl.ds(ebase, n_edges)],
                        sbuf.at[pl.ds(0, n_edges)])
        pltpu.sync_copy(dst_hbm.at[pl.ds(ebase, n_edges)],
                        dbuf.at[pl.ds(0, n_edges)])
        if n_edges < nf * FS:
            for k in range(n_edges // 16, (nf * FS) // 16):
                sbuf[pl.ds(k * 16, 16)] = jnp.zeros((16,), jnp.int32)
                dbuf[pl.ds(k * 16, 16)] = jnp.full((16,), NP, jnp.int32)

        def cp(k, carry):
            dv = dbuf[pl.ds(k * 16, 16)]
            dfire[k // 8, pl.ds((k % 8) * 16, 16)] = dv
            return carry
        lax.fori_loop(0, (nf * FS) // 16, cp, 0)

        cps = [pltpu.async_copy(x_hbm.at[sbuf.at[pl.ds(f * FS, FS)]],
                                vals.at[pl.ds(f * FS, FS)], gsem)
               for f in range(nf)]
        for d in cps:
            d.wait()
        scs = [pltpu.async_copy(vals.at[pl.ds(f * FS, FS)],
                                agg_sp.at[dfire.at[f]], ssem, add=True)
               for f in range(nf)]
        for d in scs:
            d.wait()

    def blk(b, carry):
        process(tile_base + b * EB, EB, EB // FS)
        return carry
    lax.fori_loop(0, NBLK0, blk, 0)
    process(tile_base + NBLK0 * EB, TAIL0, TAIL0_PAD // FS)

    plsc.subcore_barrier()

    @pl.when(c == 0)
    def _():
        pltpu.sync_copy(agg_sp.at[pl.ds(s * 6400, 6400)],
                        out0_hbm.at[pl.ds(s * 6400, 6400)])

    @pl.when(c == 1)
    def _():
        pltpu.sync_copy(agg_sp.at[pl.ds(s * 6400, 6400)],
                        out1_hbm.at[pl.ds(s * 6400, 6400)])


def _l0_agg(xf, src, dst):
    kfn = pl.kernel(
        _l0_body,
        out_type=[jax.ShapeDtypeStruct((NP,), jnp.float32),
                  jax.ShapeDtypeStruct((NP,), jnp.float32)],
        mesh=plsc.VectorSubcoreMesh(**_VMESH),
        compiler_params=_SC_PARAMS,
        scratch_types=[
            pltpu.VMEM((EB,), jnp.int32),
            pltpu.VMEM((EB,), jnp.int32),
            pltpu.VMEM((16, FS), jnp.int32),
            pltpu.VMEM((EB,), jnp.float32),
            pltpu.VMEM((6400,), jnp.float32),
            pltpu.VMEM_SHARED((NP + FS,), jnp.float32),
            pltpu.SemaphoreType.DMA,
            pltpu.SemaphoreType.DMA,
        ],
    )
    return kfn(src, dst, xf)


TRB = 125
TRNB = 25


def _tr_body(h_hbm, ht_hbm, hin, lsem, osem):
    c = lax.axis_index("c")
    s = lax.axis_index("s")
    w = s * 2 + c
    base = w * (TRNB * TRB)

    def load(b, q):
        return pltpu.async_copy(h_hbm.at[pl.ds(base + b * TRB, TRB), :],
                                hin.at[q], lsem)

    pending_load = load(0, 0)
    pending_outs = []
    for b in range(TRNB):
        q = b % 2
        pending_load.wait()
        for d in pending_outs:
            d.wait()
        if b + 1 < TRNB:
            pending_load = load(b + 1, 1 - q)
        pending_outs = [
            pltpu.async_copy(hin.at[q].at[:, pl.ds(g * GW, GW)],
                             ht_hbm.at[pl.ds(g * NP + base + b * TRB, TRB), :],
                             osem)
            for g in range(NG)]
    for d in pending_outs:
        d.wait()


def _transpose(h):
    kfn = pl.kernel(
        _tr_body,
        out_type=jax.ShapeDtypeStruct((NG * NP, GW), jnp.float32),
        mesh=plsc.VectorSubcoreMesh(**_VMESH),
        compiler_params=_SC_PARAMS,
        scratch_types=[
            pltpu.VMEM((2, TRB, H), jnp.float32),
            pltpu.SemaphoreType.DMA,
            pltpu.SemaphoreType.DMA,
        ],
    )
    return kfn(h)


EBA = 512
NF = EBA // FS
NBLK = 196
ETP = NBLK * EBA
EP = 16 * ETP


def _agg_body(srcg_hbm, dst2_hbm, ht_hbm, agg_hbm,
              sbuf, dbuf, rows, zbuf, agg_sp, esem, gsem, ssem):
    c = lax.axis_index("c")
    s = lax.axis_index("s")

    def _z(r, carry):
        zbuf[r, :] = jnp.zeros((GW,), jnp.float32)
        return carry
    lax.fori_loop(0, 400, _z, 0)

    tile_base = s * ETP

    def issue_loads(b, goff):
        q = b % 4
        pltpu.async_copy(srcg_hbm.at[pl.ds(goff + tile_base + b * EBA, EBA)],
                         sbuf.at[q], esem)
        pltpu.async_copy(dst2_hbm.at[pl.ds(tile_base + b * EBA, EBA)],
                         dbuf.at[q], esem)

    def wait_loads(b):
        q = b % 4
        pltpu.make_async_copy(srcg_hbm.at[pl.ds(tile_base, EBA)],
                              sbuf.at[q], esem).wait()
        pltpu.make_async_copy(dst2_hbm.at[pl.ds(tile_base, EBA)],
                              dbuf.at[q], esem).wait()

    def issue_gathers(b):
        ql, qr = b % 4, b % 2
        pltpu.async_copy(ht_hbm.at[sbuf.at[ql]], rows.at[qr], gsem)

    def wait_gathers(b):
        ql, qr = b % 4, b % 2
        pltpu.make_async_copy(ht_hbm.at[sbuf.at[ql]], rows.at[qr], gsem).wait()

    def issue_scatters(b):
        ql, qr = b % 4, b % 2
        pltpu.async_copy(rows.at[qr], agg_sp.at[dbuf.at[ql]], ssem, add=True)

    def wait_scatters(b):
        ql, qr = b % 4, b % 2
        pltpu.make_async_copy(rows.at[qr], agg_sp.at[dbuf.at[ql]], ssem).wait()

    for gk in range(NG // 2):
        g = c * (NG // 2) + gk
        goff = g * EP

        for m in range(16):
            pltpu.sync_copy(zbuf, agg_sp.at[pl.ds(s * 6400 + m * 400, 400), :])
        plsc.subcore_barrier()

        issue_loads(0, goff)
        issue_loads(1, goff)
        wait_loads(0)
        issue_gathers(0)
        issue_loads(2, goff)
        wait_loads(1)
        issue_gathers(1)
        issue_loads(3, goff)
        wait_gathers(0)
        issue_scatters(0)

        def quad(i, carry):
            for j in range(4):
                b = 4 * i + 2 + j
                wait_loads(b)
                wait_scatters(b - 2)
                issue_gathers(b)
                issue_loads(b + 2, goff)
                wait_gathers(b - 1)
                issue_scatters(b - 1)
            return carry
        lax.fori_loop(0, 48, quad, 0)

        for b in (194, 195):
            wait_loads(b)
            wait_scatters(b - 2)
            issue_gathers(b)
            wait_gathers(b - 1)
            issue_scatters(b - 1)
        wait_gathers(195)
        issue_scatters(195)
        wait_scatters(194)
        wait_scatters(195)

        plsc.subcore_barrier()
        pltpu.sync_copy(agg_sp.at[pl.ds(s * 6400, 6400), :],
                        agg_hbm.at[pl.ds(s * 6400, 6400), pl.ds(g * GW, GW)])
        plsc.subcore_barrier()


def _edge_agg(srcg, dst2, ht):
    kfn = pl.kernel(
        _agg_body,
        out_type=jax.ShapeDtypeStruct((NP, H), jnp.float32),
        mesh=plsc.VectorSubcoreMesh(**_VMESH),
        compiler_params=_SC_PARAMS,
        scratch_types=[
            pltpu.VMEM((4, EBA), jnp.int32),
            pltpu.VMEM((4, EBA), jnp.int32),
            pltpu.VMEM((2, EBA, GW), jnp.float32),
            pltpu.VMEM((400, GW), jnp.float32),
            pltpu.VMEM_SHARED((NP + 8, GW), jnp.float32),
            pltpu.SemaphoreType.DMA,
            pltpu.SemaphoreType.DMA,
            pltpu.SemaphoreType.DMA,
        ],
    )
    return kfn(srcg, dst2, ht)


def _gelu(v):
    return 0.5 * v * (1.0 + jax.lax.erf(v * 0.7071067811865476))


def _mlp_body(h_ref, agg_ref, w1_ref, b1_ref, w2_ref, b2_ref, o_ref):
    u = h_ref[...] + agg_ref[...]
    t = _gelu(jnp.dot(u, w1_ref[...], preferred_element_type=jnp.float32) + b1_ref[...])
    o_ref[...] = _gelu(jnp.dot(t, w2_ref[...], preferred_element_type=jnp.float32) + b2_ref[...])


def _mlp(h, agg, W1, b1, W2, b2):
    n, d = h.shape
    return pl.pallas_call(
        _mlp_body,
        grid=(n // BN,),
        in_specs=[
            pl.BlockSpec((BN, d), lambda i: (i, 0)),
            pl.BlockSpec((BN, d), lambda i: (i, 0)),
            pl.BlockSpec((d, H), lambda i: (0, 0)),
            pl.BlockSpec((1, H), lambda i: (0, 0)),
            pl.BlockSpec((H, H), lambda i: (0, 0)),
            pl.BlockSpec((1, H), lambda i: (0, 0)),
        ],
        out_specs=pl.BlockSpec((BN, H), lambda i: (i, 0)),
        out_shape=jax.ShapeDtypeStruct((n, H), jnp.float32),
    )(h, agg, W1, b1.reshape(1, H), W2, b2.reshape(1, H))


def _mlp0_body(u_ref, w1_ref, b1_ref, w2_ref, b2_ref, o_ref):
    t = _gelu(u_ref[...] * w1_ref[...] + b1_ref[...])
    o_ref[...] = _gelu(jnp.dot(t, w2_ref[...], preferred_element_type=jnp.float32) + b2_ref[...])


def _mlp0(u, W1, b1, W2, b2):
    n = u.shape[0]
    return pl.pallas_call(
        _mlp0_body,
        grid=(n // BN,),
        in_specs=[
            pl.BlockSpec((BN, 1), lambda i: (i, 0)),
            pl.BlockSpec((1, H), lambda i: (0, 0)),
            pl.BlockSpec((1, H), lambda i: (0, 0)),
            pl.BlockSpec((H, H), lambda i: (0, 0)),
            pl.BlockSpec((1, H), lambda i: (0, 0)),
        ],
        out_specs=pl.BlockSpec((BN, H), lambda i: (i, 0)),
        out_shape=jax.ShapeDtypeStruct((n, H), jnp.float32),
    )(u, W1, b1.reshape(1, H), W2, b2.reshape(1, H))


def _final_body(h_ref, batch_ref, wf_ref, bf_ref, o_ref):
    i = pl.program_id(0)

    @pl.when(i == 0)
    def _():
        o_ref[...] = jnp.zeros_like(o_ref)

    out = jnp.dot(h_ref[...], wf_ref[...], preferred_element_type=jnp.float32) + bf_ref[0, 0]
    gid = jax.lax.broadcasted_iota(jnp.int32, (1, G), 1)
    onehot = (batch_ref[...] == gid).astype(jnp.float32)
    o_ref[...] += jnp.sum(onehot * out, axis=0, keepdims=True)


def _final_pool(h, batch, Wf, bf):
    n = h.shape[0]
    pooled = pl.pallas_call(
        _final_body,
        grid=(n // BN,),
        in_specs=[
            pl.BlockSpec((BN, H), lambda i: (i, 0)),
            pl.BlockSpec((BN, 1), lambda i: (i, 0)),
            pl.BlockSpec((H, 1), lambda i: (0, 0)),
            pl.BlockSpec((1, 1), lambda i: (0, 0)),
        ],
        out_specs=pl.BlockSpec((1, G), lambda i: (0, 0)),
        out_shape=jax.ShapeDtypeStruct((1, G), jnp.float32),
    )(h, batch.reshape(n, 1), Wf, bf.reshape(1, 1))
    return pooled.reshape(G, 1)


def kernel(x, edge_index, batch,
           W1_0, b1_0, W2_0, b2_0,
           W1_1, b1_1, W2_1, b2_1,
           W1_2, b1_2, W2_2, b2_2,
           Wf, bf):
    src = edge_index[0]
    dst = edge_index[1]
    pad = EP - E
    src_pp = jnp.concatenate([src, jnp.zeros((pad,), jnp.int32)])
    dst_pp = jnp.concatenate([dst, jnp.full((pad,), NP, jnp.int32)])
    srcg = (src_pp[None, :]
            + (jnp.arange(NG, dtype=jnp.int32) * NP)[:, None]).reshape(-1)
    dst2 = dst_pp
    xf = x.reshape(-1)

    p0, p1 = _l0_agg(xf, src, dst)
    u = x + (p0 + p1)[:N].reshape(N, 1)
    h = _mlp0(u, W1_0, b1_0, W2_0, b2_0)

    for (W1, b1, W2, b2) in ((W1_1, b1_1, W2_1, b2_1), (W1_2, b1_2, W2_2, b2_2)):
        ht = _transpose(h)
        agg = _edge_agg(srcg, dst2, ht)
        h = _mlp(h, agg, W1, b1, W2, b2)

    return _final_pool(h, batch, Wf, bf)
